# baseline (speedup 1.0000x reference)
import jax
import jax.numpy as jnp
from jax import lax
from jax.experimental import pallas as pl
from jax.experimental.pallas import tpu as pltpu
from jax.experimental.pallas import tpu_sc as plsc

N = 10000
K = 5000
D = 128
ZPAD = 16
NSRC = N + 16
KSCAN = 320
SGROUPS = KSCAN // 16
R = 320
NCHUNK = 4
CH = R // NCHUNK
HZROWS = 5120

NBLK = 50
BR = N // NBLK


def _sc_body(h_hbm, idxp_hbm, fill_hbm, zrows_hbm, out_hbm, hz_hbm,
             idx_v, dests_v, vals_v, gidx_v, rows_v, srcs_sp, sem):
    c = lax.axis_index("c")
    s = lax.axis_index("s")
    wid = c * 16 + s
    lanes = lax.iota(jnp.int32, 16)

    pltpu.sync_copy(idxp_hbm.at[pl.ds(s * KSCAN, KSCAN + 8)], idx_v)

    @pl.when(s == 0)
    def _init():
        pltpu.sync_copy(fill_hbm, srcs_sp)
        pltpu.sync_copy(zrows_hbm, hz_hbm.at[pl.ds(0, ZPAD)])

    hb = jnp.minimum(s * 320, K - 320)
    pltpu.sync_copy(h_hbm.at[pl.ds(hb, 320)], rows_v)
    pltpu.sync_copy(rows_v, hz_hbm.at[pl.ds(ZPAD + hb, 320)])

    kk0 = s * KSCAN + ZPAD + lanes

    for chunk in range(NCHUNK):
        for gi in range(CH // 16):
            off = chunk * CH + gi * 16
            v = idx_v[pl.ds(off, 16)]
            vn = plsc.load_gather(idx_v, [off + 1 + lanes])
            delta = jnp.where(v != vn, kk0 + off, 0)
            dests_v[chunk, pl.ds(gi * 16, 16)] = v
            vals_v[chunk, pl.ds(gi * 16, 16)] = delta

    for j in range(NCHUNK):
        pltpu.sync_copy(vals_v.at[j], srcs_sp.at[dests_v.at[j]], add=True)

    plsc.subcore_barrier()

    rowbase = jnp.minimum(wid * R, N - R)
    for j in range(NCHUNK):
        pltpu.sync_copy(srcs_sp.at[pl.ds(rowbase + j * CH, CH)], gidx_v.at[j])
    copies = [
        pltpu.async_copy(
            hz_hbm.at[gidx_v.at[j]],
            rows_v.at[pl.ds(j * CH, CH)],
            sem,
        )
        for j in range(NCHUNK)
    ]
    for cp in copies:
        cp.wait()
    pltpu.sync_copy(rows_v, out_hbm.at[pl.ds(rowbase, R)])


_sc_unpool = pl.kernel(
    _sc_body,
    out_type=(
        jax.ShapeDtypeStruct((N, D), jnp.float32),
        jax.ShapeDtypeStruct((HZROWS, D), jnp.float32),
    ),
    mesh=plsc.VectorSubcoreMesh(core_axis_name="c", subcore_axis_name="s"),
    compiler_params=pltpu.CompilerParams(needs_layout_passes=False),
    scratch_types=[
        pltpu.VMEM((KSCAN + 8,), jnp.int32),
        pltpu.VMEM((NCHUNK, CH), jnp.int32),
        pltpu.VMEM((NCHUNK, CH), jnp.int32),
        pltpu.VMEM((NCHUNK, CH), jnp.int32),
        pltpu.VMEM((R, D), jnp.float32),
        pltpu.VMEM_SHARED((NSRC,), jnp.int32),
        pltpu.SemaphoreType.DMA,
    ],
)


def _copy_body(g_ref, g_out_ref):
    g_out_ref[...] = g_ref[...]


def kernel(g, h, pre_h, idx):
    idx32 = idx.astype(jnp.int32)
    idxp = jnp.concatenate(
        [idx32, jnp.full((16 * SGROUPS * 16 + 8 + 16 - K,), N, jnp.int32)]
    )
    fill = jnp.zeros((NSRC,), jnp.int32)
    zrows = jnp.zeros((ZPAD, D), jnp.float32)
    new_h, _ = _sc_unpool(h, idxp, fill, zrows)
    g_out = pl.pallas_call(
        _copy_body,
        grid=(NBLK,),
        out_shape=jax.ShapeDtypeStruct((N, N), jnp.float32),
        in_specs=[pl.BlockSpec((BR, N), lambda i: (i, 0))],
        out_specs=pl.BlockSpec((BR, N), lambda i: (i, 0)),
    )(g)
    return (g_out, new_h)


# speedup vs baseline: 1.1046x; 1.1046x over previous
import jax
import jax.numpy as jnp
from jax import lax
from jax.experimental import pallas as pl
from jax.experimental.pallas import tpu as pltpu
from jax.experimental.pallas import tpu_sc as plsc

N = 10000
K = 5000
D = 128
NS = N + 16
KSCAN = 320
NCHUNK = 4
CH = KSCAN // NCHUNK
HALF = N // 2

NBLK = 50
BR = N // NBLK


def _sc_body(h_hbm, idxp_hbm, zblk_hbm, out_hbm,
             idx_v, dests_v, rows_v, sem):
    c = lax.axis_index("c")
    s = lax.axis_index("s")
    lanes = lax.iota(jnp.int32, 16)

    kb = jnp.minimum(s * KSCAN, K - KSCAN)
    pltpu.sync_copy(idxp_hbm.at[pl.ds(kb, KSCAN + 8)], idx_v)

    zb = c * HALF + jnp.minimum(s * KSCAN, HALF - KSCAN)
    pltpu.sync_copy(zblk_hbm, rows_v)
    pltpu.sync_copy(rows_v, out_hbm.at[pl.ds(zb, KSCAN)])

    half_lo = c * HALF
    for chunk in range(NCHUNK):
        for gi in range(CH // 16):
            off = chunk * CH + gi * 16
            v = idx_v[pl.ds(off, 16)]
            vn = plsc.load_gather(idx_v, [off + 1 + lanes])
            vlo = v - half_lo
            keep = (v != vn) & (vlo >= 0) & (vlo < HALF)
            dests_v[chunk, pl.ds(gi * 16, 16)] = jnp.where(keep, v, N)

    pltpu.sync_copy(h_hbm.at[pl.ds(kb, KSCAN)], rows_v)

    plsc.subcore_barrier()

    copies = [
        pltpu.async_copy(
            rows_v.at[pl.ds(j * CH, CH)],
            out_hbm.at[dests_v.at[j]],
            sem,
        )
        for j in range(NCHUNK)
    ]
    for cp in copies:
        cp.wait()


_sc_unpool = pl.kernel(
    _sc_body,
    out_type=jax.ShapeDtypeStruct((NS, D), jnp.float32),
    mesh=plsc.VectorSubcoreMesh(core_axis_name="c", subcore_axis_name="s"),
    compiler_params=pltpu.CompilerParams(needs_layout_passes=False),
    scratch_types=[
        pltpu.VMEM((KSCAN + 8,), jnp.int32),
        pltpu.VMEM((NCHUNK, CH), jnp.int32),
        pltpu.VMEM((KSCAN, D), jnp.float32),
        pltpu.SemaphoreType.DMA,
    ],
)


def _copy_body(g_ref, g_out_ref):
    g_out_ref[...] = g_ref[...]


def kernel(g, h, pre_h, idx):
    idx32 = idx.astype(jnp.int32)
    idxp = jnp.concatenate(
        [idx32, jnp.full((16 * KSCAN + 24 - K,), N, jnp.int32)]
    )
    zblk = jnp.zeros((KSCAN, D), jnp.float32)
    scratch = _sc_unpool(h, idxp, zblk)
    new_h = lax.slice(scratch, (0, 0), (N, D))
    g_out = pl.pallas_call(
        _copy_body,
        grid=(NBLK,),
        out_shape=jax.ShapeDtypeStruct((N, N), jnp.float32),
        in_specs=[pl.BlockSpec((BR, N), lambda i: (i, 0))],
        out_specs=pl.BlockSpec((BR, N), lambda i: (i, 0)),
    )(g)
    return (g_out, new_h)


# speedup vs baseline: 1.8958x; 1.7162x over previous
import jax
import jax.numpy as jnp
from jax import lax
from jax.experimental import pallas as pl
from jax.experimental.pallas import tpu as pltpu
from jax.experimental.pallas import tpu_sc as plsc

N = 10000
K = 5000
D = 128
NS = N + 520
KSCAN = 320
NCHUNK = 4
CH = KSCAN // NCHUNK
HALF = N // 2

NBLK = 50
BR = N // NBLK


def _sc_body(h_hbm, idxp_hbm, zblk_hbm, out_hbm,
             idx_v, dests_v, rows_v, sem):
    c = lax.axis_index("c")
    s = lax.axis_index("s")
    lanes = lax.iota(jnp.int32, 16)
    dumpvec = N + (c * 16 + s) * 16 + lanes

    kb = jnp.minimum(s * KSCAN, K - KSCAN)
    pltpu.sync_copy(idxp_hbm.at[pl.ds(kb, KSCAN + 8)], idx_v)

    zb = c * HALF + jnp.minimum(s * KSCAN, HALF - KSCAN)
    pltpu.sync_copy(zblk_hbm, rows_v)
    pltpu.sync_copy(rows_v, out_hbm.at[pl.ds(zb, KSCAN)])

    half_lo = c * HALF
    for chunk in range(NCHUNK):
        for gi in range(CH // 16):
            off = chunk * CH + gi * 16
            v = idx_v[pl.ds(off, 16)]
            vn = plsc.load_gather(idx_v, [off + 1 + lanes])
            vlo = v - half_lo
            keep = (v != vn) & (vlo >= 0) & (vlo < HALF)
            dests_v[chunk, pl.ds(gi * 16, 16)] = jnp.where(keep, v, dumpvec)

    pltpu.sync_copy(h_hbm.at[pl.ds(kb, KSCAN)], rows_v)

    plsc.subcore_barrier()

    copies = [
        pltpu.async_copy(
            rows_v.at[pl.ds(j * CH, CH)],
            out_hbm.at[dests_v.at[j]],
            sem,
        )
        for j in range(NCHUNK)
    ]
    for cp in copies:
        cp.wait()


_sc_unpool = pl.kernel(
    _sc_body,
    out_type=jax.ShapeDtypeStruct((NS, D), jnp.float32),
    mesh=plsc.VectorSubcoreMesh(core_axis_name="c", subcore_axis_name="s"),
    compiler_params=pltpu.CompilerParams(needs_layout_passes=False),
    scratch_types=[
        pltpu.VMEM((KSCAN + 8,), jnp.int32),
        pltpu.VMEM((NCHUNK, CH), jnp.int32),
        pltpu.VMEM((KSCAN, D), jnp.float32),
        pltpu.SemaphoreType.DMA,
    ],
)


def _copy_body(g_ref, g_out_ref):
    g_out_ref[...] = g_ref[...]


def kernel(g, h, pre_h, idx):
    idx32 = idx.astype(jnp.int32)
    idxp = jnp.concatenate(
        [idx32, jnp.full((16 * KSCAN + 24 - K,), N, jnp.int32)]
    )
    zblk = jnp.zeros((KSCAN, D), jnp.float32)
    scratch = _sc_unpool(h, idxp, zblk)
    new_h = lax.slice(scratch, (0, 0), (N, D))
    g_out = pl.pallas_call(
        _copy_body,
        grid=(NBLK,),
        out_shape=jax.ShapeDtypeStruct((N, N), jnp.float32),
        in_specs=[pl.BlockSpec((BR, N), lambda i: (i, 0))],
        out_specs=pl.BlockSpec((BR, N), lambda i: (i, 0)),
    )(g)
    return (g_out, new_h)


# speedup vs baseline: 1.9047x; 1.0047x over previous
import jax
import jax.numpy as jnp
from jax import lax
from jax.experimental import pallas as pl
from jax.experimental.pallas import tpu as pltpu
from jax.experimental.pallas import tpu_sc as plsc

N = 10000
K = 5000
D = 128
NS = N + 520
KSCAN = 320
NCHUNK = 4
CH = KSCAN // NCHUNK
HALF = N // 2

NBLK = 50
BR = N // NBLK


def _sc_body(h_hbm, idxp_hbm, zblk_hbm, out_hbm,
             idx_v, dests_v, rows_v, sem):
    c = lax.axis_index("c")
    s = lax.axis_index("s")
    lanes = lax.iota(jnp.int32, 16)
    dumpvec = N + (c * 16 + s) * 16 + lanes

    kb = jnp.minimum(s * KSCAN, K - KSCAN)
    pltpu.sync_copy(idxp_hbm.at[pl.ds(kb, KSCAN)], idx_v.at[pl.ds(0, KSCAN)])

    @pl.when(s < 15)
    def _stage_tail():
        pltpu.sync_copy(
            idxp_hbm.at[pl.ds(kb + KSCAN, 16)], idx_v.at[pl.ds(KSCAN, 16)]
        )

    @pl.when(s == 15)
    def _sentinel_tail():
        idx_v[pl.ds(KSCAN, 16)] = jnp.full((16,), N, jnp.int32)

    zb = c * HALF + jnp.minimum(s * KSCAN, HALF - KSCAN)
    pltpu.sync_copy(zblk_hbm, rows_v)
    pltpu.sync_copy(rows_v, out_hbm.at[pl.ds(zb, KSCAN)])

    half_lo = c * HALF
    for chunk in range(NCHUNK):
        for gi in range(CH // 16):
            off = chunk * CH + gi * 16
            v = idx_v[pl.ds(off, 16)]
            vn = plsc.load_gather(idx_v, [off + 1 + lanes])
            vlo = v - half_lo
            keep = (v != vn) & (vlo >= 0) & (vlo < HALF)
            dests_v[chunk, pl.ds(gi * 16, 16)] = jnp.where(keep, v, dumpvec)

    pltpu.sync_copy(h_hbm.at[pl.ds(kb, KSCAN)], rows_v)

    plsc.subcore_barrier()

    copies = [
        pltpu.async_copy(
            rows_v.at[pl.ds(j * CH, CH)],
            out_hbm.at[dests_v.at[j]],
            sem,
        )
        for j in range(NCHUNK)
    ]
    for cp in copies:
        cp.wait()


_sc_unpool = pl.kernel(
    _sc_body,
    out_type=jax.ShapeDtypeStruct((NS, D), jnp.float32),
    mesh=plsc.VectorSubcoreMesh(core_axis_name="c", subcore_axis_name="s"),
    compiler_params=pltpu.CompilerParams(needs_layout_passes=False),
    scratch_types=[
        pltpu.VMEM((KSCAN + 16,), jnp.int32),
        pltpu.VMEM((NCHUNK, CH), jnp.int32),
        pltpu.VMEM((KSCAN, D), jnp.float32),
        pltpu.SemaphoreType.DMA,
    ],
)


def _copy_body(g_ref, g_out_ref):
    g_out_ref[...] = g_ref[...]


def kernel(g, h, pre_h, idx):
    idx32 = idx.astype(jnp.int32)
    zblk = jnp.zeros((KSCAN, D), jnp.float32)
    scratch = _sc_unpool(h, idx32, zblk)
    new_h = lax.slice(scratch, (0, 0), (N, D))
    g_out = pl.pallas_call(
        _copy_body,
        grid=(NBLK,),
        out_shape=jax.ShapeDtypeStruct((N, N), jnp.float32),
        in_specs=[pl.BlockSpec((BR, N), lambda i: (i, 0))],
        out_specs=pl.BlockSpec((BR, N), lambda i: (i, 0)),
    )(g)
    return (g_out, new_h)
